# speedup vs baseline: 1.3026x; 1.3026x over previous
import functools

import jax
import jax.numpy as jnp
from jax import lax
from jax.experimental import pallas as pl
from jax.experimental.pallas import tpu as pltpu
from jax.experimental.pallas import tpu_sc as plsc

_info = plsc.get_sparse_core_info()
_NC, _NS, _L = _info.num_cores, _info.num_subcores, _info.num_lanes
_NW = _NC * _NS

_B = 16384
_D = 64
_BPW = _B // _NW
_CH = 128
_NCHUNK = _BPW // _CH


def _film_body(x_hbm, idx_hbm, film_hbm, out_hbm,
               idx_v, x_v, rows0, rows1, sem_x, sem_g0, sem_g1, sem_st):
    wid = lax.axis_index("s") * _NC + lax.axis_index("c")
    base = wid * _BPW

    cx = pltpu.async_copy(x_hbm.at[pl.ds(base, _BPW)], x_v, sem_x)
    for c in range(_NCHUNK):
        pltpu.sync_copy(idx_hbm.at[pl.ds(base + c * _CH, _CH)], idx_v.at[c])

    rows = (rows0, rows1)
    sems = (sem_g0, sem_g1)
    descs = [None] * _NCHUNK
    descs[0] = pltpu.async_copy(film_hbm.at[idx_v.at[0]], rows[0], sems[0])
    cx.wait()

    st_descs = []
    for c in range(_NCHUNK):
        if c + 1 < _NCHUNK:
            descs[c + 1] = pltpu.async_copy(
                film_hbm.at[idx_v.at[c + 1]], rows[(c + 1) % 2],
                sems[(c + 1) % 2])
        descs[c].wait()
        rb = rows[c % 2]
        xoff = c * _CH

        def row_body(r, _, rb=rb, xoff=xoff):
            for j in range(_D // _L):
                g = rb[r, pl.ds(j * _L, _L)]
                b = rb[r, pl.ds(_D + j * _L, _L)]
                xx = x_v[xoff + r, pl.ds(j * _L, _L)]
                x_v[xoff + r, pl.ds(j * _L, _L)] = g * xx + b
            return _

        lax.fori_loop(0, _CH, row_body, 0)
        st_descs.append(pltpu.async_copy(
            x_v.at[pl.ds(xoff, _CH)],
            out_hbm.at[pl.ds(base + xoff, _CH)], sem_st))

    for d in st_descs:
        d.wait()


@jax.jit
def _film(x, idx32, film):
    mesh = plsc.VectorSubcoreMesh(core_axis_name="c", subcore_axis_name="s")
    return pl.kernel(
        _film_body,
        out_type=jax.ShapeDtypeStruct((_B, _D), jnp.float32),
        mesh=mesh,
        scratch_types=[
            pltpu.VMEM((_NCHUNK, _CH), jnp.int32),
            pltpu.VMEM((_BPW, _D), jnp.float32),
            pltpu.VMEM((_CH, 2 * _D), jnp.float32),
            pltpu.VMEM((_CH, 2 * _D), jnp.float32),
            pltpu.SemaphoreType.DMA,
            pltpu.SemaphoreType.DMA,
            pltpu.SemaphoreType.DMA,
            pltpu.SemaphoreType.DMA,
        ],
    )(x, idx32, film)


def kernel(x, cell_line, film):
    idx32 = cell_line.astype(jnp.int32)
    out = _film(x, idx32, film)
    return (out, cell_line)
